# Initial kernel scaffold; baseline (speedup 1.0000x reference)
import jax
import jax.numpy as jnp
from jax.experimental import pallas as pl


def kernel(center, context, negatives, W_in, W_out):
    raise NotImplementedError("write your pallas kernel here")


# speedup vs baseline: 2.7686x; 2.7686x over previous
import functools

import jax
import jax.numpy as jnp
from jax import lax
from jax.experimental import pallas as pl
from jax.experimental.pallas import tpu as pltpu
from jax.experimental.pallas import tpu_sc as plsc

_VOCAB = 100000
_DIM = 64
_B = 16384
_K = 5

_NC = 2
_NS = 16
_NW = _NC * _NS
_BPW = _B // _NW
_S = 128
_NCHUNK = _BPW // _S
_G = _S // 16


@functools.cache
def _make_sc_logits():
    mesh = plsc.VectorSubcoreMesh(core_axis_name="c", subcore_axis_name="s")

    @functools.partial(
        pl.kernel,
        mesh=mesh,
        compiler_params=pltpu.CompilerParams(
            needs_layout_passes=False, use_tc_tiling_on_sc=False),
        out_type=jax.ShapeDtypeStruct((6, _B), jnp.float32),
        scratch_types=[
            pltpu.VMEM((_S,), jnp.int32),
            pltpu.VMEM((_S,), jnp.int32),
            pltpu.VMEM((_K * _S,), jnp.int32),
            pltpu.VMEM((_S, _DIM), jnp.float32),
            pltpu.VMEM((_S, _DIM), jnp.float32),
            pltpu.VMEM((_K * _S, _DIM), jnp.float32),
            pltpu.VMEM((6, _S), jnp.float32),
            pltpu.SemaphoreType.DMA,
            pltpu.SemaphoreType.DMA,
            pltpu.SemaphoreType.DMA,
        ],
    )
    def sc_logits(center_hbm, ctx_hbm, neg_hbm, win_hbm, wout_hbm, out_hbm,
                  idxc, idxx, idxn, crows, xrows, nrows, lbuf,
                  sem0, sem1, sem2):
        wid = lax.axis_index("s") * _NC + lax.axis_index("c")
        base = wid * _BPW
        iota = lax.iota(jnp.int32, 16)

        def chunk_body(t, carry):
            cbase = pl.multiple_of(base + t * _S, _S)
            pltpu.sync_copy(center_hbm.at[pl.ds(cbase, _S)], idxc)
            pltpu.sync_copy(ctx_hbm.at[pl.ds(cbase, _S)], idxx)
            pltpu.sync_copy(neg_hbm.at[pl.ds(cbase * _K, _K * _S)], idxn)
            cp0 = pltpu.async_copy(win_hbm.at[idxc], crows, sem0)
            cp1 = pltpu.async_copy(wout_hbm.at[idxx], xrows, sem1)
            cp2 = pltpu.async_copy(wout_hbm.at[idxn], nrows, sem2)
            cp0.wait()
            cp1.wait()
            cp2.wait()

            def g_body(g, carry2):
                s0 = pl.multiple_of(g * 16, 16)
                accs = [jnp.zeros((16,), jnp.float32) for _ in range(6)]
                for l in range(16):
                    s = s0 + l
                    lane = iota == l
                    cvs = [crows[s, pl.ds(k * 16, 16)] for k in range(_DIM // 16)]
                    for j in range(6):
                        if j == 0:
                            rvs = [xrows[s, pl.ds(k * 16, 16)]
                                   for k in range(_DIM // 16)]
                        else:
                            rvs = [nrows[s * _K + (j - 1), pl.ds(k * 16, 16)]
                                   for k in range(_DIM // 16)]
                        p = cvs[0] * rvs[0]
                        for k in range(1, _DIM // 16):
                            p = p + cvs[k] * rvs[k]
                        r = jnp.sum(p)
                        accs[j] = jnp.where(lane, r, accs[j])
                for j in range(6):
                    lbuf[j, pl.ds(s0, 16)] = accs[j]
                return carry2

            lax.fori_loop(0, _G, g_body, 0)
            for j in range(6):
                pltpu.sync_copy(lbuf.at[j], out_hbm.at[j, pl.ds(cbase, _S)])
            return carry

        lax.fori_loop(0, _NCHUNK, chunk_body, 0)

    return sc_logits


def _bce_body(x_ref, o_ref):
    x = x_ref[...]
    lbl = (lax.broadcasted_iota(jnp.int32, x.shape, 0) == 0).astype(jnp.float32)
    v = jnp.maximum(x, 0.0) - x * lbl + jnp.log(1.0 + jnp.exp(-jnp.abs(x)))
    o_ref[0, 0] = jnp.sum(v) / (6.0 * _B)


def kernel(center, context, negatives, W_in, W_out):
    cen = center.astype(jnp.int32)
    ctx = context.reshape(_B).astype(jnp.int32)
    neg = negatives.reshape(_B * _K).astype(jnp.int32)
    logits = _make_sc_logits()(cen, ctx, neg, W_in, W_out)
    loss = pl.pallas_call(
        _bce_body,
        out_shape=jax.ShapeDtypeStruct((1, 1), jnp.float32),
        out_specs=pl.BlockSpec(memory_space=pltpu.SMEM),
    )(logits)
    return loss[0, 0]
